# Initial kernel scaffold; baseline (speedup 1.0000x reference)
import jax
import jax.numpy as jnp
from jax.experimental import pallas as pl


def kernel(input_ids, table, W, b):
    raise NotImplementedError("write your pallas kernel here")


# speedup vs baseline: 2.5071x; 2.5071x over previous
import functools

import jax
import jax.numpy as jnp
from jax import lax
from jax.experimental import pallas as pl
from jax.experimental.pallas import tpu as pltpu
from jax.experimental.pallas import tpu_sc as plsc


def _proj_body(t_ref, w_ref, b_ref, o_ref):
    o_ref[...] = (
        jnp.dot(t_ref[...], w_ref[...], preferred_element_type=jnp.float32)
        + b_ref[...]
    )


def _project_table(table, W, b2d, block_rows):
    V, H = table.shape
    D = W.shape[1]
    grid = (V // block_rows,)
    return pl.pallas_call(
        _proj_body,
        grid=grid,
        in_specs=[
            pl.BlockSpec((block_rows, H), lambda i: (i, 0)),
            pl.BlockSpec((H, D), lambda i: (0, 0)),
            pl.BlockSpec((1, D), lambda i: (0, 0)),
        ],
        out_specs=pl.BlockSpec((block_rows, D), lambda i: (i, 0)),
        out_shape=jax.ShapeDtypeStruct((V, D), jnp.float32),
    )(table, W, b2d)


def _make_sc_gather(V, D, B, n_workers, chunk):
    b_per_w = B // n_workers
    n_chunks = b_per_w // chunk
    mesh = plsc.VectorSubcoreMesh(core_axis_name="c", subcore_axis_name="s")

    @functools.partial(
        pl.kernel,
        mesh=mesh,
        out_type=jax.ShapeDtypeStruct((B, D), jnp.float32),
        scratch_types=[
            pltpu.VMEM((b_per_w,), jnp.int32),
            pltpu.VMEM((chunk, D), jnp.float32),
            pltpu.VMEM((chunk, D), jnp.float32),
            pltpu.SemaphoreType.DMA,
            pltpu.SemaphoreType.DMA,
        ],
    )
    def gather_kernel(tab_hbm, idx_hbm, out_hbm, idx_v, buf0, buf1, sem0, sem1):
        n_cores = 2
        wid = lax.axis_index("s") * n_cores + lax.axis_index("c")
        base = wid * b_per_w
        pltpu.sync_copy(idx_hbm.at[pl.ds(base, b_per_w)], idx_v)

        bufs = (buf0, buf1)
        sems = (sem0, sem1)
        gathers = []
        for i in range(n_chunks):
            g = pltpu.async_copy(
                tab_hbm.at[idx_v.at[pl.ds(i * chunk, chunk)]],
                bufs[i % 2],
                sems[i % 2],
            )
            gathers.append(g)
            if i >= 1:
                gathers[i - 1].wait()
                pltpu.sync_copy(
                    bufs[(i - 1) % 2],
                    out_hbm.at[pl.ds(base + (i - 1) * chunk, chunk)],
                )
        gathers[n_chunks - 1].wait()
        pltpu.sync_copy(
            bufs[(n_chunks - 1) % 2],
            out_hbm.at[pl.ds(base + (n_chunks - 1) * chunk, chunk)],
        )

    return gather_kernel


def kernel(input_ids, table, W, b):
    Bt, S = input_ids.shape
    V, H = table.shape
    D = W.shape[1]
    B = Bt * S

    proj = _project_table(table, W, b.reshape(1, D), block_rows=2000)

    idx = input_ids.reshape(B).astype(jnp.int32)
    n_workers = 32
    out_flat = _make_sc_gather(V, D, B, n_workers, chunk=320)(proj, idx)
    return out_flat.reshape(Bt, S, D)


# speedup vs baseline: 3.8587x; 1.5391x over previous
import functools

import jax
import jax.numpy as jnp
from jax import lax
from jax.experimental import pallas as pl
from jax.experimental.pallas import tpu as pltpu
from jax.experimental.pallas import tpu_sc as plsc


def _proj_body(t_ref, w_ref, b_ref, o_ref):
    o_ref[...] = (
        jnp.dot(t_ref[...], w_ref[...], preferred_element_type=jnp.float32)
        + b_ref[...]
    )


def _project_table(table, W, b2d, block_rows):
    V, H = table.shape
    D = W.shape[1]
    grid = (V // block_rows,)
    return pl.pallas_call(
        _proj_body,
        grid=grid,
        in_specs=[
            pl.BlockSpec((block_rows, H), lambda i: (i, 0)),
            pl.BlockSpec((H, D), lambda i: (0, 0)),
            pl.BlockSpec((1, D), lambda i: (0, 0)),
        ],
        out_specs=pl.BlockSpec((block_rows, D), lambda i: (i, 0)),
        out_shape=jax.ShapeDtypeStruct((V, D), jnp.float32),
    )(table, W, b2d)


def _make_sc_gather(V, D, B, S, n_workers, G):
    nb = B // n_workers
    T = nb // G
    assert T % 2 == 0
    mesh = plsc.VectorSubcoreMesh(core_axis_name="c", subcore_axis_name="s")

    @functools.partial(
        pl.kernel,
        mesh=mesh,
        out_type=jax.ShapeDtypeStruct((B, S, D), jnp.float32),
        scratch_types=[
            pltpu.VMEM((nb, S), jnp.int32),
            pltpu.VMEM((G, S, D), jnp.float32),
            pltpu.VMEM((G, S, D), jnp.float32),
            pltpu.SemaphoreType.DMA,
            pltpu.SemaphoreType.DMA,
            pltpu.SemaphoreType.DMA,
            pltpu.SemaphoreType.DMA,
        ],
    )
    def gather_kernel(tab_hbm, ids_hbm, out_hbm, idx_v, buf0, buf1,
                      gsem0, gsem1, wbsem0, wbsem1):
        n_cores = 2
        wid = lax.axis_index("s") * n_cores + lax.axis_index("c")
        b0 = wid * nb
        pltpu.sync_copy(ids_hbm.at[pl.ds(b0, nb)], idx_v)

        bufs = (buf0, buf1)
        gsems = (gsem0, gsem1)
        wbsems = (wbsem0, wbsem1)

        def issue_gathers(t, buf, gsem):
            for k in range(G):
                pltpu.async_copy(tab_hbm.at[idx_v.at[t * G + k]],
                                 buf.at[k], gsem)

        def drain_gathers(buf, gsem):
            for k in range(G):
                pltpu.make_async_copy(tab_hbm.at[idx_v.at[0]],
                                      buf.at[k], gsem).wait()

        def issue_writeback(t, buf, wbsem):
            pltpu.async_copy(buf, out_hbm.at[pl.ds(b0 + t * G, G)], wbsem)

        def drain_writeback(buf, wbsem):
            pltpu.make_async_copy(buf, out_hbm.at[pl.ds(b0, G)], wbsem).wait()

        issue_gathers(0, buf0, gsem0)

        def body(s, carry):
            te = 2 * s
            to = 2 * s + 1

            @pl.when(s > 0)
            def _():
                drain_writeback(buf1, wbsem1)

            issue_gathers(to, buf1, gsem1)
            drain_gathers(buf0, gsem0)
            issue_writeback(te, buf0, wbsem0)

            drain_gathers(buf1, gsem1)
            issue_writeback(to, buf1, wbsem1)

            @pl.when(s < T // 2 - 1)
            def _():
                drain_writeback(buf0, wbsem0)
                issue_gathers(te + 2, buf0, gsem0)

            return carry

        lax.fori_loop(0, T // 2, body, 0)
        drain_writeback(buf0, wbsem0)
        drain_writeback(buf1, wbsem1)

    return gather_kernel


def kernel(input_ids, table, W, b):
    Bt, S = input_ids.shape
    V, H = table.shape
    D = W.shape[1]

    proj = _project_table(table, W, b.reshape(1, D), block_rows=2000)
    ids = input_ids.astype(jnp.int32)
    return _make_sc_gather(V, D, Bt, S, n_workers=32, G=8)(proj, ids)
